# baseline (speedup 1.0000x reference)
import jax
import jax.numpy as jnp
from jax import lax
from jax.experimental import pallas as pl
from jax.experimental.pallas import tpu as pltpu
from jax.experimental.pallas import tpu_sc as plsc

B = 16384
D = 64
NC = 2
NS = 16
NW = NC * NS
BPW = B // NW


def _gather_concat_body(uid_hbm, rid_hbm, ing_hbm, ut_hbm, rt_hbm, it_hbm,
                        out_hbm, uidx_v, ridx_v, iidx_v,
                        urows_v, rrows_v, irows_v, sem):
    wid = lax.axis_index("s") * NC + lax.axis_index("c")
    base = wid * BPW
    pltpu.sync_copy(uid_hbm.at[pl.ds(base, BPW)], uidx_v)
    pltpu.sync_copy(rid_hbm.at[pl.ds(base, BPW)], ridx_v)
    pltpu.sync_copy(ing_hbm.at[pl.ds(base, BPW)], iidx_v)
    cu = pltpu.async_copy(ut_hbm.at[uidx_v], urows_v, sem)
    cr = pltpu.async_copy(rt_hbm.at[ridx_v], rrows_v, sem)
    ci = pltpu.async_copy(it_hbm.at[iidx_v], irows_v, sem)
    cu.wait()
    cr.wait()
    ci.wait()
    pltpu.sync_copy(urows_v, out_hbm.at[pl.ds(base, BPW), pl.ds(0, D)])
    pltpu.sync_copy(rrows_v, out_hbm.at[pl.ds(base, BPW), pl.ds(D, D)])
    pltpu.sync_copy(irows_v, out_hbm.at[pl.ds(base, BPW), pl.ds(2 * D, D)])


def kernel(uid, rid, ing, user_table, recipe_table, ingredient_table):
    mesh = plsc.VectorSubcoreMesh(core_axis_name="c", subcore_axis_name="s")
    f = pl.kernel(
        _gather_concat_body,
        mesh=mesh,
        compiler_params=pltpu.CompilerParams(use_tc_tiling_on_sc=False),
        out_type=jax.ShapeDtypeStruct((B, 3 * D), jnp.float32),
        scratch_types=[
            pltpu.VMEM((BPW,), jnp.int32),
            pltpu.VMEM((BPW,), jnp.int32),
            pltpu.VMEM((BPW,), jnp.int32),
            pltpu.VMEM((BPW, D), jnp.float32),
            pltpu.VMEM((BPW, D), jnp.float32),
            pltpu.VMEM((BPW, D), jnp.float32),
            pltpu.SemaphoreType.DMA,
        ],
    )
    return f(uid, rid, ing, user_table, recipe_table, ingredient_table)
